# baseline (speedup 1.0000x reference)
import functools

import jax
import jax.numpy as jnp
from jax import lax
from jax.experimental import pallas as pl
from jax.experimental.pallas import tpu as pltpu
from jax.experimental.pallas import tpu_sc as plsc

B = 16384
D = 32
H = 64
SLOT = 128
PACK = SLOT // D
NC, NS = 2, 16
NW = NC * NS
B_PER_W = B // NW
CHUNK = 128
NCHUNK = B_PER_W // CHUNK
NBUF = 3


def _sc_gather_slots(uids2d, iids2d, utab4, itab4):
    mesh = plsc.VectorSubcoreMesh(core_axis_name="c", subcore_axis_name="s")

    @functools.partial(
        pl.kernel,
        mesh=mesh,
        out_type=[
            jax.ShapeDtypeStruct((B, SLOT), jnp.float32),
            jax.ShapeDtypeStruct((B, SLOT), jnp.float32),
        ],
        scratch_types=[
            pltpu.VMEM((NCHUNK, CHUNK), jnp.int32),
            pltpu.VMEM((NCHUNK, CHUNK), jnp.int32),
            [pltpu.VMEM((CHUNK, SLOT), jnp.float32) for _ in range(NBUF)],
            [pltpu.VMEM((CHUNK, SLOT), jnp.float32) for _ in range(NBUF)],
            pltpu.SemaphoreType.DMA,
            pltpu.SemaphoreType.DMA,
            pltpu.SemaphoreType.DMA,
            pltpu.SemaphoreType.DMA,
        ],
    )
    def gather_kernel(uids, iids, utab, itab, uout, iout,
                      uidx, iidx, ubufs, ibufs, ugsem, igsem, uwsem, iwsem):
        wid = lax.axis_index("s") * NC + lax.axis_index("c")
        base = wid * B_PER_W
        row0 = wid * NCHUNK
        pltpu.sync_copy(uids.at[pl.ds(row0, NCHUNK)], uidx)
        pltpu.sync_copy(iids.at[pl.ds(row0, NCHUNK)], iidx)

        def gather(j):
            return (
                pltpu.async_copy(utab.at[uidx.at[j]], ubufs[j % NBUF], ugsem),
                pltpu.async_copy(itab.at[iidx.at[j]], ibufs[j % NBUF], igsem),
            )

        def writeback(j):
            dst = pl.ds(base + j * CHUNK, CHUNK)
            return (
                pltpu.async_copy(ubufs[j % NBUF], uout.at[dst], uwsem),
                pltpu.async_copy(ibufs[j % NBUF], iout.at[dst], iwsem),
            )

        gathers = [gather(j) for j in range(NBUF)]
        writes = []
        for j in range(NCHUNK):
            for c in gathers[j]:
                c.wait()
            writes.append(writeback(j))
            nxt = j + NBUF
            if nxt < NCHUNK:
                for c in writes[nxt - NBUF]:
                    c.wait()
                gathers.append(gather(nxt))
        for j in range(max(0, NCHUNK - NBUF + 1), NCHUNK):
            for c in writes[j]:
                c.wait()

    return gather_kernel(uids2d, iids2d, utab4, itab4)


TCH = 512


def _slotize_body(t_ref, o_ref):
    x = t_ref[...]
    parts = [x[:, 128 * j:128 * (j + 1)].T for j in range(PACK)]
    o_ref[...] = jnp.concatenate(parts, axis=1)


def _tc_slotize(tab_t, nrows):
    grid = (nrows + TCH - 1) // TCH
    return pl.pallas_call(
        _slotize_body,
        grid=(grid,),
        in_specs=[pl.BlockSpec((D, TCH), lambda i: (0, i))],
        out_specs=pl.BlockSpec((TCH // PACK, SLOT), lambda i: (i, 0)),
        out_shape=jax.ShapeDtypeStruct((grid * (TCH // PACK), SLOT), jnp.float32),
    )(tab_t)


BLK = 2048


def _mlp_body(us_ref, is_ref, ug_ref, ig_ref,
              w1u_ref, w1i_ref, b1_ref, w2_ref, b2_ref, o_ref):
    us = us_ref[...]
    it = is_ref[...]
    ug = ug_ref[...]
    ig = ig_ref[...]
    u = jnp.zeros((BLK, D), jnp.float32)
    i = jnp.zeros((BLK, D), jnp.float32)
    for k in range(PACK):
        u = jnp.where(ug == k, us[:, k * D:(k + 1) * D], u)
        i = jnp.where(ig == k, it[:, k * D:(k + 1) * D], i)
    h = (jnp.dot(u, w1u_ref[...], preferred_element_type=jnp.float32)
         + jnp.dot(i, w1i_ref[...], preferred_element_type=jnp.float32)
         + b1_ref[...])
    h = jnp.maximum(h, 0.0)
    z = jnp.sum(h * w2_ref[...], axis=1, keepdims=True) + b2_ref[...]
    o_ref[...] = jax.nn.sigmoid(z)


def _tc_mlp(uslots, islots, ugrp, igrp, w1u, w1i, b1_2d, w2_2d, b2_2d):
    return pl.pallas_call(
        _mlp_body,
        grid=(B // BLK,),
        in_specs=[
            pl.BlockSpec((BLK, SLOT), lambda i: (i, 0)),
            pl.BlockSpec((BLK, SLOT), lambda i: (i, 0)),
            pl.BlockSpec((BLK, 1), lambda i: (i, 0)),
            pl.BlockSpec((BLK, 1), lambda i: (i, 0)),
            pl.BlockSpec((D, H), lambda i: (0, 0)),
            pl.BlockSpec((D, H), lambda i: (0, 0)),
            pl.BlockSpec((1, H), lambda i: (0, 0)),
            pl.BlockSpec((1, H), lambda i: (0, 0)),
            pl.BlockSpec((1, 1), lambda i: (0, 0)),
        ],
        out_specs=pl.BlockSpec((BLK, 1), lambda i: (i, 0)),
        out_shape=jax.ShapeDtypeStruct((B, 1), jnp.float32),
    )(uslots, islots, ugrp, igrp, w1u, w1i, b1_2d, w2_2d, b2_2d)


def kernel(user_ids, item_ids, user_table, item_table, W1, b1, W2, b2):
    uids = user_ids.astype(jnp.int32)
    iids = item_ids.astype(jnp.int32)
    utab4 = _tc_slotize(user_table.T, user_table.shape[0])
    itab4 = _tc_slotize(item_table.T, item_table.shape[0])
    uids2d = ((uids // TCH) * 128 + uids % 128).reshape(B // CHUNK, CHUNK)
    iids2d = ((iids // TCH) * 128 + iids % 128).reshape(B // CHUNK, CHUNK)
    ugrp = ((uids // 128) % PACK).reshape(B, 1)
    igrp = ((iids // 128) % PACK).reshape(B, 1)
    uslots, islots = _sc_gather_slots(uids2d, iids2d, utab4, itab4)
    w1u = W1[:, :D].T
    w1i = W1[:, D:].T
    b1_2d = b1.reshape(1, H)
    w2_2d = W2.reshape(1, H)
    b2_2d = b2.reshape(1, 1)
    return _tc_mlp(uslots, islots, ugrp, igrp, w1u, w1i, b1_2d, w2_2d, b2_2d)


# speedup vs baseline: 4.1382x; 4.1382x over previous
import functools

import jax
import jax.numpy as jnp
from jax import lax
from jax.experimental import pallas as pl
from jax.experimental.pallas import tpu as pltpu
from jax.experimental.pallas import tpu_sc as plsc

B = 16384
D = 32
H = 64
SLOT = 128
PACK = SLOT // D
NC, NS = 2, 16
NW = NC * NS
B_PER_W = B // NW
CHUNK = 128
NCHUNK = B_PER_W // CHUNK
NBUF = 3


def _sc_gather_slots(uids2d, iids2d, utab4, itab4):
    mesh = plsc.VectorSubcoreMesh(core_axis_name="c", subcore_axis_name="s")

    @functools.partial(
        pl.kernel,
        mesh=mesh,
        out_type=[
            jax.ShapeDtypeStruct((B, SLOT), jnp.float32),
            jax.ShapeDtypeStruct((B, SLOT), jnp.float32),
        ],
        scratch_types=[
            pltpu.VMEM((NCHUNK, CHUNK), jnp.int32),
            pltpu.VMEM((NCHUNK, CHUNK), jnp.int32),
            [pltpu.VMEM((CHUNK, SLOT), jnp.float32) for _ in range(NBUF)],
            [pltpu.VMEM((CHUNK, SLOT), jnp.float32) for _ in range(NBUF)],
            pltpu.SemaphoreType.DMA,
            pltpu.SemaphoreType.DMA,
            pltpu.SemaphoreType.DMA,
            pltpu.SemaphoreType.DMA,
        ],
    )
    def gather_kernel(uids, iids, utab, itab, uout, iout,
                      uidx, iidx, ubufs, ibufs, ugsem, igsem, uwsem, iwsem):
        wid = lax.axis_index("s") * NC + lax.axis_index("c")
        base = wid * B_PER_W
        row0 = wid * NCHUNK
        pltpu.sync_copy(uids.at[pl.ds(row0, NCHUNK)], uidx)
        pltpu.sync_copy(iids.at[pl.ds(row0, NCHUNK)], iidx)

        def gather(j):
            return (
                pltpu.async_copy(utab.at[uidx.at[j]], ubufs[j % NBUF], ugsem),
                pltpu.async_copy(itab.at[iidx.at[j]], ibufs[j % NBUF], igsem),
            )

        def writeback(j):
            dst = pl.ds(base + j * CHUNK, CHUNK)
            return (
                pltpu.async_copy(ubufs[j % NBUF], uout.at[dst], uwsem),
                pltpu.async_copy(ibufs[j % NBUF], iout.at[dst], iwsem),
            )

        gathers = [gather(j) for j in range(NBUF)]
        writes = []
        for j in range(NCHUNK):
            for c in gathers[j]:
                c.wait()
            writes.append(writeback(j))
            nxt = j + NBUF
            if nxt < NCHUNK:
                for c in writes[nxt - NBUF]:
                    c.wait()
                gathers.append(gather(nxt))
        for j in range(max(0, NCHUNK - NBUF + 1), NCHUNK):
            for c in writes[j]:
                c.wait()

    return gather_kernel(uids2d, iids2d, utab4, itab4)


TCH = 8192


def _slotize_body(t_ref, o_ref):
    x = t_ref[...]
    for s in range(TCH // 512):
        parts = [x[:, 512 * s + 128 * j:512 * s + 128 * (j + 1)].T
                 for j in range(PACK)]
        o_ref[128 * s:128 * (s + 1), :] = jnp.concatenate(parts, axis=1)


def _tc_slotize(tab_t, nrows):
    grid = (nrows + TCH - 1) // TCH
    return pl.pallas_call(
        _slotize_body,
        grid=(grid,),
        in_specs=[pl.BlockSpec((D, TCH), lambda i: (0, i))],
        out_specs=pl.BlockSpec((TCH // PACK, SLOT), lambda i: (i, 0)),
        out_shape=jax.ShapeDtypeStruct((grid * (TCH // PACK), SLOT), jnp.float32),
    )(tab_t)


BLK = 2048


def _mlp_body(us_ref, is_ref, ug_ref, ig_ref,
              w1u_ref, w1i_ref, b1_ref, w2_ref, b2_ref, o_ref):
    us = us_ref[...]
    it = is_ref[...]
    ug = ug_ref[...]
    ig = ig_ref[...]
    u = jnp.zeros((BLK, D), jnp.float32)
    i = jnp.zeros((BLK, D), jnp.float32)
    for k in range(PACK):
        u = jnp.where(ug == k, us[:, k * D:(k + 1) * D], u)
        i = jnp.where(ig == k, it[:, k * D:(k + 1) * D], i)
    h = (jnp.dot(u, w1u_ref[...], preferred_element_type=jnp.float32)
         + jnp.dot(i, w1i_ref[...], preferred_element_type=jnp.float32)
         + b1_ref[...])
    h = jnp.maximum(h, 0.0)
    z = jnp.sum(h * w2_ref[...], axis=1, keepdims=True) + b2_ref[...]
    o_ref[...] = jax.nn.sigmoid(z)


def _tc_mlp(uslots, islots, ugrp, igrp, w1u, w1i, b1_2d, w2_2d, b2_2d):
    return pl.pallas_call(
        _mlp_body,
        grid=(B // BLK,),
        in_specs=[
            pl.BlockSpec((BLK, SLOT), lambda i: (i, 0)),
            pl.BlockSpec((BLK, SLOT), lambda i: (i, 0)),
            pl.BlockSpec((BLK, 1), lambda i: (i, 0)),
            pl.BlockSpec((BLK, 1), lambda i: (i, 0)),
            pl.BlockSpec((D, H), lambda i: (0, 0)),
            pl.BlockSpec((D, H), lambda i: (0, 0)),
            pl.BlockSpec((1, H), lambda i: (0, 0)),
            pl.BlockSpec((1, H), lambda i: (0, 0)),
            pl.BlockSpec((1, 1), lambda i: (0, 0)),
        ],
        out_specs=pl.BlockSpec((BLK, 1), lambda i: (i, 0)),
        out_shape=jax.ShapeDtypeStruct((B, 1), jnp.float32),
    )(uslots, islots, ugrp, igrp, w1u, w1i, b1_2d, w2_2d, b2_2d)


def kernel(user_ids, item_ids, user_table, item_table, W1, b1, W2, b2):
    uids = user_ids.astype(jnp.int32)
    iids = item_ids.astype(jnp.int32)
    utab4 = _tc_slotize(user_table.T, user_table.shape[0])
    itab4 = _tc_slotize(item_table.T, item_table.shape[0])
    uids2d = ((uids // TCH) * 128 + uids % 128).reshape(B // CHUNK, CHUNK)
    iids2d = ((iids // TCH) * 128 + iids % 128).reshape(B // CHUNK, CHUNK)
    ugrp = ((uids // 128) % PACK).reshape(B, 1)
    igrp = ((iids // 128) % PACK).reshape(B, 1)
    uslots, islots = _sc_gather_slots(uids2d, iids2d, utab4, itab4)
    w1u = W1[:, :D].T
    w1i = W1[:, D:].T
    b1_2d = b1.reshape(1, H)
    w2_2d = W2.reshape(1, H)
    b2_2d = b2.reshape(1, 1)
    return _tc_mlp(uslots, islots, ugrp, igrp, w1u, w1i, b1_2d, w2_2d, b2_2d)


# speedup vs baseline: 4.1400x; 1.0004x over previous
import functools

import jax
import jax.numpy as jnp
from jax import lax
from jax.experimental import pallas as pl
from jax.experimental.pallas import tpu as pltpu
from jax.experimental.pallas import tpu_sc as plsc

B = 16384
D = 32
H = 64
SLOT = 128
PACK = SLOT // D
NC, NS = 2, 16
NW = NC * NS
B_PER_W = B // NW
CHUNK = 128
NCHUNK = B_PER_W // CHUNK
NBUF = 3


def _sc_gather_slots(uids2d, iids2d, utab4, itab4):
    mesh = plsc.VectorSubcoreMesh(core_axis_name="c", subcore_axis_name="s")

    @functools.partial(
        pl.kernel,
        mesh=mesh,
        out_type=[
            jax.ShapeDtypeStruct((B, SLOT), jnp.float32),
            jax.ShapeDtypeStruct((B, SLOT), jnp.float32),
        ],
        scratch_types=[
            pltpu.VMEM((NCHUNK, CHUNK), jnp.int32),
            pltpu.VMEM((NCHUNK, CHUNK), jnp.int32),
            [pltpu.VMEM((CHUNK, SLOT), jnp.float32) for _ in range(NBUF)],
            [pltpu.VMEM((CHUNK, SLOT), jnp.float32) for _ in range(NBUF)],
            pltpu.SemaphoreType.DMA,
            pltpu.SemaphoreType.DMA,
            pltpu.SemaphoreType.DMA,
            pltpu.SemaphoreType.DMA,
        ],
    )
    def gather_kernel(uids, iids, utab, itab, uout, iout,
                      uidx, iidx, ubufs, ibufs, ugsem, igsem, uwsem, iwsem):
        wid = lax.axis_index("s") * NC + lax.axis_index("c")
        base = wid * B_PER_W
        row0 = wid * NCHUNK
        pltpu.sync_copy(uids.at[pl.ds(row0, NCHUNK)], uidx)
        pltpu.sync_copy(iids.at[pl.ds(row0, NCHUNK)], iidx)

        def gather(j):
            return (
                pltpu.async_copy(utab.at[uidx.at[j]], ubufs[j % NBUF], ugsem),
                pltpu.async_copy(itab.at[iidx.at[j]], ibufs[j % NBUF], igsem),
            )

        def writeback(j):
            dst = pl.ds(base + j * CHUNK, CHUNK)
            return (
                pltpu.async_copy(ubufs[j % NBUF], uout.at[dst], uwsem),
                pltpu.async_copy(ibufs[j % NBUF], iout.at[dst], iwsem),
            )

        gathers = [gather(j) for j in range(NBUF)]
        writes = []
        for j in range(NCHUNK):
            for c in gathers[j]:
                c.wait()
            writes.append(writeback(j))
            nxt = j + NBUF
            if nxt < NCHUNK:
                for c in writes[nxt - NBUF]:
                    c.wait()
                gathers.append(gather(nxt))
        for j in range(max(0, NCHUNK - NBUF + 1), NCHUNK):
            for c in writes[j]:
                c.wait()

    return gather_kernel(uids2d, iids2d, utab4, itab4)


TCH = 8192


def _slotize_body(t_ref, o_ref):
    for s in range(TCH // 512):
        parts = [t_ref[:, 512 * s + 128 * j:512 * s + 128 * (j + 1)].T
                 for j in range(PACK)]
        o_ref[128 * s:128 * (s + 1), :] = jnp.concatenate(parts, axis=1)


def _tc_slotize(tab_t, nrows):
    grid = (nrows + TCH - 1) // TCH
    return pl.pallas_call(
        _slotize_body,
        grid=(grid,),
        in_specs=[pl.BlockSpec((D, TCH), lambda i: (0, i))],
        out_specs=pl.BlockSpec((TCH // PACK, SLOT), lambda i: (i, 0)),
        out_shape=jax.ShapeDtypeStruct((grid * (TCH // PACK), SLOT), jnp.float32),
    )(tab_t)


BLK = 2048


def _mlp_body(us_ref, is_ref, ug_ref, ig_ref,
              w1u_ref, w1i_ref, b1_ref, w2_ref, b2_ref, o_ref):
    us = us_ref[...]
    it = is_ref[...]
    ug = ug_ref[...]
    ig = ig_ref[...]
    u = jnp.zeros((BLK, D), jnp.float32)
    i = jnp.zeros((BLK, D), jnp.float32)
    for k in range(PACK):
        u = jnp.where(ug == k, us[:, k * D:(k + 1) * D], u)
        i = jnp.where(ig == k, it[:, k * D:(k + 1) * D], i)
    h = (jnp.dot(u, w1u_ref[...], preferred_element_type=jnp.float32)
         + jnp.dot(i, w1i_ref[...], preferred_element_type=jnp.float32)
         + b1_ref[...])
    h = jnp.maximum(h, 0.0)
    z = jnp.sum(h * w2_ref[...], axis=1, keepdims=True) + b2_ref[...]
    o_ref[...] = jax.nn.sigmoid(z)


def _tc_mlp(uslots, islots, ugrp, igrp, w1u, w1i, b1_2d, w2_2d, b2_2d):
    return pl.pallas_call(
        _mlp_body,
        grid=(B // BLK,),
        in_specs=[
            pl.BlockSpec((BLK, SLOT), lambda i: (i, 0)),
            pl.BlockSpec((BLK, SLOT), lambda i: (i, 0)),
            pl.BlockSpec((BLK, 1), lambda i: (i, 0)),
            pl.BlockSpec((BLK, 1), lambda i: (i, 0)),
            pl.BlockSpec((D, H), lambda i: (0, 0)),
            pl.BlockSpec((D, H), lambda i: (0, 0)),
            pl.BlockSpec((1, H), lambda i: (0, 0)),
            pl.BlockSpec((1, H), lambda i: (0, 0)),
            pl.BlockSpec((1, 1), lambda i: (0, 0)),
        ],
        out_specs=pl.BlockSpec((BLK, 1), lambda i: (i, 0)),
        out_shape=jax.ShapeDtypeStruct((B, 1), jnp.float32),
    )(uslots, islots, ugrp, igrp, w1u, w1i, b1_2d, w2_2d, b2_2d)


def kernel(user_ids, item_ids, user_table, item_table, W1, b1, W2, b2):
    uids = user_ids.astype(jnp.int32)
    iids = item_ids.astype(jnp.int32)
    utab4 = _tc_slotize(user_table.T, user_table.shape[0])
    itab4 = _tc_slotize(item_table.T, item_table.shape[0])
    uids2d = ((uids // 512) * 128 + uids % 128).reshape(B // CHUNK, CHUNK)
    iids2d = ((iids // 512) * 128 + iids % 128).reshape(B // CHUNK, CHUNK)
    ugrp = ((uids // 128) % PACK).reshape(B, 1)
    igrp = ((iids // 128) % PACK).reshape(B, 1)
    uslots, islots = _sc_gather_slots(uids2d, iids2d, utab4, itab4)
    w1u = W1[:, :D].T
    w1i = W1[:, D:].T
    b1_2d = b1.reshape(1, H)
    w2_2d = W2.reshape(1, H)
    b2_2d = b2.reshape(1, 1)
    return _tc_mlp(uslots, islots, ugrp, igrp, w1u, w1i, b1_2d, w2_2d, b2_2d)


# speedup vs baseline: 4.2285x; 1.0214x over previous
import functools

import jax
import jax.numpy as jnp
from jax import lax
from jax.experimental import pallas as pl
from jax.experimental.pallas import tpu as pltpu
from jax.experimental.pallas import tpu_sc as plsc

B = 16384
D = 32
H = 64
SLOT = 128
PACK = SLOT // D
NC, NS = 2, 16
NW = NC * NS
B_PER_W = B // NW
CHUNK = 128
NCHUNK = B_PER_W // CHUNK
NBUF = 3


def _sc_gather_slots(uids2d, iids2d, utab4, itab4):
    mesh = plsc.VectorSubcoreMesh(core_axis_name="c", subcore_axis_name="s")

    @functools.partial(
        pl.kernel,
        mesh=mesh,
        out_type=[
            jax.ShapeDtypeStruct((B, SLOT), jnp.float32),
            jax.ShapeDtypeStruct((B, SLOT), jnp.float32),
        ],
        scratch_types=[
            pltpu.VMEM((NCHUNK, CHUNK), jnp.int32),
            pltpu.VMEM((NCHUNK, CHUNK), jnp.int32),
            [pltpu.VMEM((CHUNK, SLOT), jnp.float32) for _ in range(NBUF)],
            [pltpu.VMEM((CHUNK, SLOT), jnp.float32) for _ in range(NBUF)],
            pltpu.SemaphoreType.DMA,
            pltpu.SemaphoreType.DMA,
            pltpu.SemaphoreType.DMA,
            pltpu.SemaphoreType.DMA,
        ],
    )
    def gather_kernel(uids, iids, utab, itab, uout, iout,
                      uidx, iidx, ubufs, ibufs, ugsem, igsem, uwsem, iwsem):
        wid = lax.axis_index("s") * NC + lax.axis_index("c")
        base = wid * B_PER_W
        row0 = wid * NCHUNK
        pltpu.sync_copy(uids.at[pl.ds(row0, NCHUNK)], uidx)
        pltpu.sync_copy(iids.at[pl.ds(row0, NCHUNK)], iidx)

        def gather(j):
            return (
                pltpu.async_copy(utab.at[uidx.at[j]], ubufs[j % NBUF], ugsem),
                pltpu.async_copy(itab.at[iidx.at[j]], ibufs[j % NBUF], igsem),
            )

        def writeback(j):
            dst = pl.ds(base + j * CHUNK, CHUNK)
            return (
                pltpu.async_copy(ubufs[j % NBUF], uout.at[dst], uwsem),
                pltpu.async_copy(ibufs[j % NBUF], iout.at[dst], iwsem),
            )

        gathers = [gather(j) for j in range(NBUF)]
        writes = []
        for j in range(NCHUNK):
            for c in gathers[j]:
                c.wait()
            writes.append(writeback(j))
            nxt = j + NBUF
            if nxt < NCHUNK:
                for c in writes[nxt - NBUF]:
                    c.wait()
                gathers.append(gather(nxt))
        for j in range(max(0, NCHUNK - NBUF + 1), NCHUNK):
            for c in writes[j]:
                c.wait()

    return gather_kernel(uids2d, iids2d, utab4, itab4)


TCH = 32768


def _slotize_body(t_ref, o_ref):
    for s in range(TCH // 512):
        parts = [t_ref[:, 512 * s + 128 * j:512 * s + 128 * (j + 1)].T
                 for j in range(PACK)]
        o_ref[128 * s:128 * (s + 1), :] = jnp.concatenate(parts, axis=1)


def _tc_slotize(tab_t, nrows):
    grid = (nrows + TCH - 1) // TCH
    return pl.pallas_call(
        _slotize_body,
        grid=(grid,),
        in_specs=[pl.BlockSpec((D, TCH), lambda i: (0, i))],
        out_specs=pl.BlockSpec((TCH // PACK, SLOT), lambda i: (i, 0)),
        out_shape=jax.ShapeDtypeStruct((grid * (TCH // PACK), SLOT), jnp.float32),
    )(tab_t)


BLK = 2048


def _mlp_body(us_ref, is_ref, ug_ref, ig_ref,
              w1u_ref, w1i_ref, b1_ref, w2_ref, b2_ref, o_ref):
    us = us_ref[...]
    it = is_ref[...]
    ug = ug_ref[...]
    ig = ig_ref[...]
    u = jnp.zeros((BLK, D), jnp.float32)
    i = jnp.zeros((BLK, D), jnp.float32)
    for k in range(PACK):
        u = jnp.where(ug == k, us[:, k * D:(k + 1) * D], u)
        i = jnp.where(ig == k, it[:, k * D:(k + 1) * D], i)
    h = (jnp.dot(u, w1u_ref[...], preferred_element_type=jnp.float32)
         + jnp.dot(i, w1i_ref[...], preferred_element_type=jnp.float32)
         + b1_ref[...])
    h = jnp.maximum(h, 0.0)
    z = jnp.sum(h * w2_ref[...], axis=1, keepdims=True) + b2_ref[...]
    o_ref[...] = jax.nn.sigmoid(z)


def _tc_mlp(uslots, islots, ugrp, igrp, w1u, w1i, b1_2d, w2_2d, b2_2d):
    return pl.pallas_call(
        _mlp_body,
        grid=(B // BLK,),
        in_specs=[
            pl.BlockSpec((BLK, SLOT), lambda i: (i, 0)),
            pl.BlockSpec((BLK, SLOT), lambda i: (i, 0)),
            pl.BlockSpec((BLK, 1), lambda i: (i, 0)),
            pl.BlockSpec((BLK, 1), lambda i: (i, 0)),
            pl.BlockSpec((D, H), lambda i: (0, 0)),
            pl.BlockSpec((D, H), lambda i: (0, 0)),
            pl.BlockSpec((1, H), lambda i: (0, 0)),
            pl.BlockSpec((1, H), lambda i: (0, 0)),
            pl.BlockSpec((1, 1), lambda i: (0, 0)),
        ],
        out_specs=pl.BlockSpec((BLK, 1), lambda i: (i, 0)),
        out_shape=jax.ShapeDtypeStruct((B, 1), jnp.float32),
    )(uslots, islots, ugrp, igrp, w1u, w1i, b1_2d, w2_2d, b2_2d)


def kernel(user_ids, item_ids, user_table, item_table, W1, b1, W2, b2):
    uids = user_ids.astype(jnp.int32)
    iids = item_ids.astype(jnp.int32)
    utab4 = _tc_slotize(user_table.T, user_table.shape[0])
    itab4 = _tc_slotize(item_table.T, item_table.shape[0])
    uids2d = ((uids // 512) * 128 + uids % 128).reshape(B // CHUNK, CHUNK)
    iids2d = ((iids // 512) * 128 + iids % 128).reshape(B // CHUNK, CHUNK)
    ugrp = ((uids // 128) % PACK).reshape(B, 1)
    igrp = ((iids // 128) % PACK).reshape(B, 1)
    uslots, islots = _sc_gather_slots(uids2d, iids2d, utab4, itab4)
    w1u = W1[:, :D].T
    w1i = W1[:, D:].T
    b1_2d = b1.reshape(1, H)
    w2_2d = W2.reshape(1, H)
    b2_2d = b2.reshape(1, 1)
    return _tc_mlp(uslots, islots, ugrp, igrp, w1u, w1i, b1_2d, w2_2d, b2_2d)


# speedup vs baseline: 9.5294x; 2.2536x over previous
import functools

import jax
import jax.numpy as jnp
from jax import lax
from jax.experimental import pallas as pl
from jax.experimental.pallas import tpu as pltpu
from jax.experimental.pallas import tpu_sc as plsc

B = 16384
D = 32
H = 64
SLOT = 128
PACK = SLOT // D
NC, NS = 2, 16
NW = NC * NS
B_PER_W = B // NW
CHUNK = 128
NCHUNK = B_PER_W // CHUNK
NBUF = 3


def _sc_gather_slots(uids2d, iids2d, utab4, itab4):
    mesh = plsc.VectorSubcoreMesh(core_axis_name="c", subcore_axis_name="s")

    @functools.partial(
        pl.kernel,
        mesh=mesh,
        out_type=[
            jax.ShapeDtypeStruct((B, SLOT), jnp.float32),
            jax.ShapeDtypeStruct((B, SLOT), jnp.float32),
        ],
        scratch_types=[
            pltpu.VMEM((NCHUNK, CHUNK), jnp.int32),
            pltpu.VMEM((NCHUNK, CHUNK), jnp.int32),
            [pltpu.VMEM((CHUNK, SLOT), jnp.float32) for _ in range(NBUF)],
            [pltpu.VMEM((CHUNK, SLOT), jnp.float32) for _ in range(NBUF)],
            pltpu.SemaphoreType.DMA,
            pltpu.SemaphoreType.DMA,
            pltpu.SemaphoreType.DMA,
            pltpu.SemaphoreType.DMA,
        ],
    )
    def gather_kernel(uids, iids, utab, itab, uout, iout,
                      uidx, iidx, ubufs, ibufs, ugsem, igsem, uwsem, iwsem):
        wid = lax.axis_index("s") * NC + lax.axis_index("c")
        base = wid * B_PER_W
        row0 = wid * NCHUNK
        pltpu.sync_copy(uids.at[pl.ds(row0, NCHUNK)], uidx)
        pltpu.sync_copy(iids.at[pl.ds(row0, NCHUNK)], iidx)

        def gather(j):
            return (
                pltpu.async_copy(utab.at[uidx.at[j]], ubufs[j % NBUF], ugsem),
                pltpu.async_copy(itab.at[iidx.at[j]], ibufs[j % NBUF], igsem),
            )

        def writeback(j):
            dst = pl.ds(base + j * CHUNK, CHUNK)
            return (
                pltpu.async_copy(ubufs[j % NBUF], uout.at[dst], uwsem),
                pltpu.async_copy(ibufs[j % NBUF], iout.at[dst], iwsem),
            )

        gathers = [gather(j) for j in range(NBUF)]
        writes = []
        for j in range(NCHUNK):
            for c in gathers[j]:
                c.wait()
            writes.append(writeback(j))
            nxt = j + NBUF
            if nxt < NCHUNK:
                for c in writes[nxt - NBUF]:
                    c.wait()
                gathers.append(gather(nxt))
        for j in range(max(0, NCHUNK - NBUF + 1), NCHUNK):
            for c in writes[j]:
                c.wait()

    return gather_kernel(uids2d, iids2d, utab4, itab4)


TCH = 32768


def _slotize_body(t_ref, o_ref):
    for s in range(TCH // 512):
        stacked = jnp.concatenate(
            [t_ref[:, 512 * s + 128 * j:512 * s + 128 * (j + 1)]
             for j in range(PACK)], axis=0)
        o_ref[128 * s:128 * (s + 1), :] = stacked.T


def _tc_slotize(tab_t, nrows):
    grid = (nrows + TCH - 1) // TCH
    return pl.pallas_call(
        _slotize_body,
        grid=(grid,),
        in_specs=[pl.BlockSpec((D, TCH), lambda i: (0, i))],
        out_specs=pl.BlockSpec((TCH // PACK, SLOT), lambda i: (i, 0)),
        out_shape=jax.ShapeDtypeStruct((grid * (TCH // PACK), SLOT), jnp.float32),
    )(tab_t)


BLK = 2048


def _mlp_body(us_ref, is_ref, ug_ref, ig_ref,
              w1u_ref, w1i_ref, b1_ref, w2_ref, b2_ref, o_ref):
    us = us_ref[...]
    it = is_ref[...]
    ug = ug_ref[...]
    ig = ig_ref[...]
    u = jnp.zeros((BLK, D), jnp.float32)
    i = jnp.zeros((BLK, D), jnp.float32)
    for k in range(PACK):
        u = jnp.where(ug == k, us[:, k * D:(k + 1) * D], u)
        i = jnp.where(ig == k, it[:, k * D:(k + 1) * D], i)
    h = (jnp.dot(u, w1u_ref[...], preferred_element_type=jnp.float32)
         + jnp.dot(i, w1i_ref[...], preferred_element_type=jnp.float32)
         + b1_ref[...])
    h = jnp.maximum(h, 0.0)
    z = jnp.sum(h * w2_ref[...], axis=1, keepdims=True) + b2_ref[...]
    o_ref[...] = jax.nn.sigmoid(z)


def _tc_mlp(uslots, islots, ugrp, igrp, w1u, w1i, b1_2d, w2_2d, b2_2d):
    return pl.pallas_call(
        _mlp_body,
        grid=(B // BLK,),
        in_specs=[
            pl.BlockSpec((BLK, SLOT), lambda i: (i, 0)),
            pl.BlockSpec((BLK, SLOT), lambda i: (i, 0)),
            pl.BlockSpec((BLK, 1), lambda i: (i, 0)),
            pl.BlockSpec((BLK, 1), lambda i: (i, 0)),
            pl.BlockSpec((D, H), lambda i: (0, 0)),
            pl.BlockSpec((D, H), lambda i: (0, 0)),
            pl.BlockSpec((1, H), lambda i: (0, 0)),
            pl.BlockSpec((1, H), lambda i: (0, 0)),
            pl.BlockSpec((1, 1), lambda i: (0, 0)),
        ],
        out_specs=pl.BlockSpec((BLK, 1), lambda i: (i, 0)),
        out_shape=jax.ShapeDtypeStruct((B, 1), jnp.float32),
    )(uslots, islots, ugrp, igrp, w1u, w1i, b1_2d, w2_2d, b2_2d)


def kernel(user_ids, item_ids, user_table, item_table, W1, b1, W2, b2):
    uids = user_ids.astype(jnp.int32)
    iids = item_ids.astype(jnp.int32)
    utab4 = _tc_slotize(user_table.T, user_table.shape[0])
    itab4 = _tc_slotize(item_table.T, item_table.shape[0])
    uids2d = ((uids // 512) * 128 + uids % 128).reshape(B // CHUNK, CHUNK)
    iids2d = ((iids // 512) * 128 + iids % 128).reshape(B // CHUNK, CHUNK)
    ugrp = ((uids // 128) % PACK).reshape(B, 1)
    igrp = ((iids // 128) % PACK).reshape(B, 1)
    uslots, islots = _sc_gather_slots(uids2d, iids2d, utab4, itab4)
    w1u = W1[:, :D].T
    w1i = W1[:, D:].T
    b1_2d = b1.reshape(1, H)
    w2_2d = W2.reshape(1, H)
    b2_2d = b2.reshape(1, 1)
    return _tc_mlp(uslots, islots, ugrp, igrp, w1u, w1i, b1_2d, w2_2d, b2_2d)


# speedup vs baseline: 10.5081x; 1.1027x over previous
import functools

import jax
import jax.numpy as jnp
from jax import lax
from jax.experimental import pallas as pl
from jax.experimental.pallas import tpu as pltpu
from jax.experimental.pallas import tpu_sc as plsc

B = 16384
D = 32
H = 64
SLOT = 128
PACK = SLOT // D
NC, NS = 2, 16
NW = NC * NS
B_PER_W = B // NW
CHUNK = 128
NCHUNK = B_PER_W // CHUNK
NBUF = 3


def _sc_gather_slots(uids2d, iids2d, utab4, itab4):
    mesh = plsc.VectorSubcoreMesh(core_axis_name="c", subcore_axis_name="s")

    @functools.partial(
        pl.kernel,
        mesh=mesh,
        out_type=[
            jax.ShapeDtypeStruct((B, SLOT), jnp.float32),
            jax.ShapeDtypeStruct((B, SLOT), jnp.float32),
        ],
        scratch_types=[
            pltpu.VMEM((NCHUNK, CHUNK), jnp.int32),
            pltpu.VMEM((NCHUNK, CHUNK), jnp.int32),
            [pltpu.VMEM((CHUNK, SLOT), jnp.float32) for _ in range(NBUF)],
            [pltpu.VMEM((CHUNK, SLOT), jnp.float32) for _ in range(NBUF)],
            pltpu.SemaphoreType.DMA,
            pltpu.SemaphoreType.DMA,
            pltpu.SemaphoreType.DMA,
            pltpu.SemaphoreType.DMA,
        ],
    )
    def gather_kernel(uids, iids, utab, itab, uout, iout,
                      uidx, iidx, ubufs, ibufs, ugsem, igsem, uwsem, iwsem):
        wid = lax.axis_index("s") * NC + lax.axis_index("c")
        base = wid * B_PER_W
        row0 = wid * NCHUNK
        pltpu.sync_copy(uids.at[pl.ds(row0, NCHUNK)], uidx)
        pltpu.sync_copy(iids.at[pl.ds(row0, NCHUNK)], iidx)

        def gather(j):
            return (
                pltpu.async_copy(utab.at[uidx.at[j]], ubufs[j % NBUF], ugsem),
                pltpu.async_copy(itab.at[iidx.at[j]], ibufs[j % NBUF], igsem),
            )

        def writeback(j):
            dst = pl.ds(base + j * CHUNK, CHUNK)
            return (
                pltpu.async_copy(ubufs[j % NBUF], uout.at[dst], uwsem),
                pltpu.async_copy(ibufs[j % NBUF], iout.at[dst], iwsem),
            )

        gathers = [gather(j) for j in range(NBUF)]
        writes = []
        for j in range(NCHUNK):
            for c in gathers[j]:
                c.wait()
            writes.append(writeback(j))
            nxt = j + NBUF
            if nxt < NCHUNK:
                for c in writes[nxt - NBUF]:
                    c.wait()
                gathers.append(gather(nxt))
        for j in range(max(0, NCHUNK - NBUF + 1), NCHUNK):
            for c in writes[j]:
                c.wait()

    return gather_kernel(uids2d, iids2d, utab4, itab4)


TCH = 32768


def _slotize_body(t_ref, o_ref):
    for s in range(TCH // 512):
        stacked = jnp.concatenate(
            [t_ref[:, 512 * s + 128 * j:512 * s + 128 * (j + 1)]
             for j in range(PACK)], axis=0)
        o_ref[128 * s:128 * (s + 1), :] = stacked.T


def _tc_slotize(tab_t, nrows):
    grid = (nrows + TCH - 1) // TCH
    return pl.pallas_call(
        _slotize_body,
        grid=(grid,),
        in_specs=[pl.BlockSpec((D, TCH), lambda i: (0, i))],
        out_specs=pl.BlockSpec((TCH // PACK, SLOT), lambda i: (i, 0)),
        out_shape=jax.ShapeDtypeStruct((grid * (TCH // PACK), SLOT), jnp.float32),
    )(tab_t)


BLK = 2048


def _mlp_body(us_ref, is_ref, ids_ref,
              w1u_ref, w1i_ref, b1_ref, w2_ref, b2_ref, o_ref):
    lane_grp = jax.lax.broadcasted_iota(jnp.int32, (BLK, SLOT), 1) // D
    ug = (ids_ref[:, 0:1] // 128) % PACK
    ig = (ids_ref[:, 1:2] // 128) % PACK
    u = jnp.where(lane_grp == ug, us_ref[...], 0.0)
    i = jnp.where(lane_grp == ig, is_ref[...], 0.0)
    h = (jnp.dot(u, w1u_ref[...], preferred_element_type=jnp.float32)
         + jnp.dot(i, w1i_ref[...], preferred_element_type=jnp.float32)
         + b1_ref[...])
    h = jnp.maximum(h, 0.0)
    z = jnp.sum(h * w2_ref[...], axis=1, keepdims=True) + b2_ref[...]
    o_ref[...] = jax.nn.sigmoid(z)


def _tc_mlp(uslots, islots, ids2, w1u4, w1i4, b1_2d, w2_2d, b2_2d):
    return pl.pallas_call(
        _mlp_body,
        grid=(B // BLK,),
        in_specs=[
            pl.BlockSpec((BLK, SLOT), lambda i: (i, 0)),
            pl.BlockSpec((BLK, SLOT), lambda i: (i, 0)),
            pl.BlockSpec((BLK, 2), lambda i: (i, 0)),
            pl.BlockSpec((SLOT, H), lambda i: (0, 0)),
            pl.BlockSpec((SLOT, H), lambda i: (0, 0)),
            pl.BlockSpec((1, H), lambda i: (0, 0)),
            pl.BlockSpec((1, H), lambda i: (0, 0)),
            pl.BlockSpec((1, 1), lambda i: (0, 0)),
        ],
        out_specs=pl.BlockSpec((BLK, 1), lambda i: (i, 0)),
        out_shape=jax.ShapeDtypeStruct((B, 1), jnp.float32),
    )(uslots, islots, ids2, w1u4, w1i4, b1_2d, w2_2d, b2_2d)


def kernel(user_ids, item_ids, user_table, item_table, W1, b1, W2, b2):
    uids = user_ids.astype(jnp.int32)
    iids = item_ids.astype(jnp.int32)
    utab4 = _tc_slotize(user_table.T, user_table.shape[0])
    itab4 = _tc_slotize(item_table.T, item_table.shape[0])
    uids2d = ((uids // 512) * 128 + uids % 128).reshape(B // CHUNK, CHUNK)
    iids2d = ((iids // 512) * 128 + iids % 128).reshape(B // CHUNK, CHUNK)
    ids2 = jnp.concatenate([uids.reshape(B, 1), iids.reshape(B, 1)], axis=1)
    uslots, islots = _sc_gather_slots(uids2d, iids2d, utab4, itab4)
    w1u4 = jnp.tile(W1[:, :D].T, (PACK, 1))
    w1i4 = jnp.tile(W1[:, D:].T, (PACK, 1))
    b1_2d = b1.reshape(1, H)
    w2_2d = W2.reshape(1, H)
    b2_2d = b2.reshape(1, 1)
    return _tc_mlp(uslots, islots, ids2, w1u4, w1i4, b1_2d, w2_2d, b2_2d)


# speedup vs baseline: 10.9205x; 1.0392x over previous
import functools

import jax
import jax.numpy as jnp
from jax import lax
from jax.experimental import pallas as pl
from jax.experimental.pallas import tpu as pltpu
from jax.experimental.pallas import tpu_sc as plsc

B = 16384
D = 32
H = 64
SLOT = 128
PACK = SLOT // D
NC, NS = 2, 16
NW = NC * NS
B_PER_W = B // NW
CHUNK = 128
NCHUNK = B_PER_W // CHUNK
NBUF = 3


def _sc_gather_slots(uids2d, iids2d, utab4, itab4):
    mesh = plsc.VectorSubcoreMesh(core_axis_name="c", subcore_axis_name="s")

    @functools.partial(
        pl.kernel,
        mesh=mesh,
        out_type=[
            jax.ShapeDtypeStruct((B, SLOT), jnp.float32),
            jax.ShapeDtypeStruct((B, SLOT), jnp.float32),
        ],
        scratch_types=[
            pltpu.VMEM((NCHUNK, CHUNK), jnp.int32),
            pltpu.VMEM((NCHUNK, CHUNK), jnp.int32),
            [pltpu.VMEM((CHUNK, SLOT), jnp.float32) for _ in range(NBUF)],
            [pltpu.VMEM((CHUNK, SLOT), jnp.float32) for _ in range(NBUF)],
            pltpu.SemaphoreType.DMA,
            pltpu.SemaphoreType.DMA,
            pltpu.SemaphoreType.DMA,
            pltpu.SemaphoreType.DMA,
        ],
    )
    def gather_kernel(uids, iids, utab, itab, uout, iout,
                      uidx, iidx, ubufs, ibufs, ugsem, igsem, uwsem, iwsem):
        wid = lax.axis_index("s") * NC + lax.axis_index("c")
        base = wid * B_PER_W
        row0 = wid * NCHUNK
        pltpu.sync_copy(uids.at[pl.ds(row0, NCHUNK)], uidx)
        pltpu.sync_copy(iids.at[pl.ds(row0, NCHUNK)], iidx)

        def gather(j):
            return (
                pltpu.async_copy(utab.at[uidx.at[j]], ubufs[j % NBUF], ugsem),
                pltpu.async_copy(itab.at[iidx.at[j]], ibufs[j % NBUF], igsem),
            )

        def writeback(j):
            dst = pl.ds(base + j * CHUNK, CHUNK)
            return (
                pltpu.async_copy(ubufs[j % NBUF], uout.at[dst], uwsem),
                pltpu.async_copy(ibufs[j % NBUF], iout.at[dst], iwsem),
            )

        gathers = [gather(j) for j in range(NBUF)]
        writes = []
        for j in range(NCHUNK):
            for c in gathers[j]:
                c.wait()
            writes.append(writeback(j))
            nxt = j + NBUF
            if nxt < NCHUNK:
                for c in writes[nxt - NBUF]:
                    c.wait()
                gathers.append(gather(nxt))
        for j in range(max(0, NCHUNK - NBUF + 1), NCHUNK):
            for c in writes[j]:
                c.wait()

    return gather_kernel(uids2d, iids2d, utab4, itab4)


TCH = 32768


def _slotize_body(t_ref, o_ref):
    for s in range(TCH // 512):
        stacked = jnp.concatenate(
            [t_ref[:, 512 * s + 128 * j:512 * s + 128 * (j + 1)]
             for j in range(PACK)], axis=0)
        o_ref[128 * s:128 * (s + 1), :] = stacked.T


def _tc_slotize(tab_t, nrows):
    grid = (nrows + TCH - 1) // TCH
    return pl.pallas_call(
        _slotize_body,
        grid=(grid,),
        in_specs=[pl.BlockSpec((D, TCH), lambda i: (0, i))],
        out_specs=pl.BlockSpec((TCH // PACK, SLOT), lambda i: (i, 0)),
        out_shape=jax.ShapeDtypeStruct((grid * (TCH // PACK), SLOT), jnp.float32),
    )(tab_t)


BLK = 2048


def _mlp_body(us_ref, is_ref, ids_ref,
              w1u_ref, w1i_ref, b1_ref, w2_ref, b2_ref, o_ref):
    lane_grp = jax.lax.broadcasted_iota(jnp.int32, (BLK, SLOT), 1) // D
    ug = (ids_ref[:, 0:1] // 128) % PACK
    ig = (ids_ref[:, 1:2] // 128) % PACK
    u = jnp.where(lane_grp == ug, us_ref[...], 0.0)
    i = jnp.where(lane_grp == ig, is_ref[...], 0.0)
    h = (jnp.dot(u, w1u_ref[...], preferred_element_type=jnp.float32)
         + jnp.dot(i, w1i_ref[...], preferred_element_type=jnp.float32)
         + b1_ref[...])
    h = jnp.maximum(h, 0.0)
    z = jax.lax.dot_general(w2_ref[...], h, (((1,), (1,)), ((), ())),
                            preferred_element_type=jnp.float32)
    o_ref[...] = jax.nn.sigmoid(z + b2_ref[...])


def _tc_mlp(uslots, islots, ids2, w1u4, w1i4, b1_2d, w2_2d, b2_2d):
    return pl.pallas_call(
        _mlp_body,
        grid=(B // BLK,),
        in_specs=[
            pl.BlockSpec((BLK, SLOT), lambda i: (i, 0)),
            pl.BlockSpec((BLK, SLOT), lambda i: (i, 0)),
            pl.BlockSpec((BLK, 2), lambda i: (i, 0)),
            pl.BlockSpec((SLOT, H), lambda i: (0, 0)),
            pl.BlockSpec((SLOT, H), lambda i: (0, 0)),
            pl.BlockSpec((1, H), lambda i: (0, 0)),
            pl.BlockSpec((1, H), lambda i: (0, 0)),
            pl.BlockSpec((1, 1), lambda i: (0, 0)),
        ],
        out_specs=pl.BlockSpec((1, BLK), lambda i: (0, i)),
        out_shape=jax.ShapeDtypeStruct((1, B), jnp.float32),
    )(uslots, islots, ids2, w1u4, w1i4, b1_2d, w2_2d, b2_2d)


def kernel(user_ids, item_ids, user_table, item_table, W1, b1, W2, b2):
    uids = user_ids.astype(jnp.int32)
    iids = item_ids.astype(jnp.int32)
    utab4 = _tc_slotize(user_table.T, user_table.shape[0])
    itab4 = _tc_slotize(item_table.T, item_table.shape[0])
    uids2d = ((uids // 512) * 128 + uids % 128).reshape(B // CHUNK, CHUNK)
    iids2d = ((iids // 512) * 128 + iids % 128).reshape(B // CHUNK, CHUNK)
    ids2 = jnp.concatenate([uids.reshape(B, 1), iids.reshape(B, 1)], axis=1)
    uslots, islots = _sc_gather_slots(uids2d, iids2d, utab4, itab4)
    w1u4 = jnp.tile(W1[:, :D].T, (PACK, 1))
    w1i4 = jnp.tile(W1[:, D:].T, (PACK, 1))
    b1_2d = b1.reshape(1, H)
    w2_2d = W2.reshape(1, H)
    b2_2d = b2.reshape(1, 1)
    out = _tc_mlp(uslots, islots, ids2, w1u4, w1i4, b1_2d, w2_2d, b2_2d)
    return out.T


# speedup vs baseline: 10.9703x; 1.0046x over previous
import functools

import jax
import jax.numpy as jnp
from jax import lax
from jax.experimental import pallas as pl
from jax.experimental.pallas import tpu as pltpu
from jax.experimental.pallas import tpu_sc as plsc

B = 16384
D = 32
H = 64
SLOT = 128
PACK = SLOT // D
NC, NS = 2, 16
NW = NC * NS
B_PER_W = B // NW
CHUNK = 128
NCHUNK = B_PER_W // CHUNK
NBUF = 3


def _sc_gather_slots(ids2d, tab4):
    mesh = plsc.VectorSubcoreMesh(core_axis_name="c", subcore_axis_name="s")

    @functools.partial(
        pl.kernel,
        mesh=mesh,
        out_type=jax.ShapeDtypeStruct((B, SLOT), jnp.float32),
        scratch_types=[
            pltpu.VMEM((NCHUNK, CHUNK), jnp.int32),
            [pltpu.VMEM((CHUNK, SLOT), jnp.float32) for _ in range(NBUF)],
            pltpu.SemaphoreType.DMA,
            pltpu.SemaphoreType.DMA,
        ],
    )
    def gather_kernel(ids, tab, out, idx, bufs, gsem, wsem):
        wid = lax.axis_index("s") * NC + lax.axis_index("c")
        base = wid * B_PER_W
        row0 = wid * NCHUNK
        pltpu.sync_copy(ids.at[pl.ds(row0, NCHUNK)], idx)

        def gather(j):
            return pltpu.async_copy(tab.at[idx.at[j]], bufs[j % NBUF], gsem)

        def writeback(j):
            dst = pl.ds(base + j * CHUNK, CHUNK)
            return pltpu.async_copy(bufs[j % NBUF], out.at[dst], wsem)

        gathers = [gather(j) for j in range(NBUF)]
        writes = []
        for j in range(NCHUNK):
            gathers[j].wait()
            writes.append(writeback(j))
            nxt = j + NBUF
            if nxt < NCHUNK:
                writes[nxt - NBUF].wait()
                gathers.append(gather(nxt))
        for j in range(max(0, NCHUNK - NBUF + 1), NCHUNK):
            writes[j].wait()

    return gather_kernel(ids2d, tab4)


TCH = 32768


def _slotize_body(t_ref, o_ref):
    for s in range(TCH // 512):
        stacked = jnp.concatenate(
            [t_ref[:, 512 * s + 128 * j:512 * s + 128 * (j + 1)]
             for j in range(PACK)], axis=0)
        o_ref[128 * s:128 * (s + 1), :] = stacked.T


def _tc_slotize(tab_t, nrows):
    grid = (nrows + TCH - 1) // TCH
    return pl.pallas_call(
        _slotize_body,
        grid=(grid,),
        in_specs=[pl.BlockSpec((D, TCH), lambda i: (0, i))],
        out_specs=pl.BlockSpec((TCH // PACK, SLOT), lambda i: (i, 0)),
        out_shape=jax.ShapeDtypeStruct((grid * (TCH // PACK), SLOT), jnp.float32),
    )(tab_t)


BLK = 2048


def _mlp_body(us_ref, is_ref, ids_ref,
              w1u_ref, w1i_ref, b1_ref, w2_ref, b2_ref, o_ref):
    lane_grp = jax.lax.broadcasted_iota(jnp.int32, (BLK, SLOT), 1) // D
    ug = (ids_ref[:, 0:1] // 128) % PACK
    ig = (ids_ref[:, 1:2] // 128) % PACK
    u = jnp.where(lane_grp == ug, us_ref[...], 0.0)
    i = jnp.where(lane_grp == ig, is_ref[...], 0.0)
    h = (jnp.dot(u, w1u_ref[...], preferred_element_type=jnp.float32)
         + jnp.dot(i, w1i_ref[...], preferred_element_type=jnp.float32)
         + b1_ref[...])
    h = jnp.maximum(h, 0.0)
    z = jax.lax.dot_general(w2_ref[...], h, (((1,), (1,)), ((), ())),
                            preferred_element_type=jnp.float32)
    o_ref[...] = jax.nn.sigmoid(z + b2_ref[...])


def _tc_mlp(uslots, islots, ids2, w1u4, w1i4, b1_2d, w2_2d, b2_2d):
    return pl.pallas_call(
        _mlp_body,
        grid=(B // BLK,),
        in_specs=[
            pl.BlockSpec((BLK, SLOT), lambda i: (i, 0)),
            pl.BlockSpec((BLK, SLOT), lambda i: (i, 0)),
            pl.BlockSpec((BLK, 2), lambda i: (i, 0)),
            pl.BlockSpec((SLOT, H), lambda i: (0, 0)),
            pl.BlockSpec((SLOT, H), lambda i: (0, 0)),
            pl.BlockSpec((1, H), lambda i: (0, 0)),
            pl.BlockSpec((1, H), lambda i: (0, 0)),
            pl.BlockSpec((1, 1), lambda i: (0, 0)),
        ],
        out_specs=pl.BlockSpec((1, BLK), lambda i: (0, i)),
        out_shape=jax.ShapeDtypeStruct((1, B), jnp.float32),
    )(uslots, islots, ids2, w1u4, w1i4, b1_2d, w2_2d, b2_2d)


def kernel(user_ids, item_ids, user_table, item_table, W1, b1, W2, b2):
    uids = user_ids.astype(jnp.int32)
    iids = item_ids.astype(jnp.int32)
    utab4 = _tc_slotize(user_table.T, user_table.shape[0])
    itab4 = _tc_slotize(item_table.T, item_table.shape[0])
    uids2d = ((uids // 512) * 128 + uids % 128).reshape(B // CHUNK, CHUNK)
    iids2d = ((iids // 512) * 128 + iids % 128).reshape(B // CHUNK, CHUNK)
    ids2 = jnp.concatenate([uids.reshape(B, 1), iids.reshape(B, 1)], axis=1)
    uslots = _sc_gather_slots(uids2d, utab4)
    islots = _sc_gather_slots(iids2d, itab4)
    w1u4 = jnp.tile(W1[:, :D].T, (PACK, 1))
    w1i4 = jnp.tile(W1[:, D:].T, (PACK, 1))
    b1_2d = b1.reshape(1, H)
    w2_2d = W2.reshape(1, H)
    b2_2d = b2.reshape(1, 1)
    out = _tc_mlp(uslots, islots, ids2, w1u4, w1i4, b1_2d, w2_2d, b2_2d)
    return out.T
